# Initial kernel scaffold; baseline (speedup 1.0000x reference)
import jax
import jax.numpy as jnp
from jax.experimental import pallas as pl


def kernel(x, edge_index, edge_weight, W1_0, W1_1, b1, W2_0, W2_1, b2, W3_0, W3_1, b3, fc1_W, fc1_b, fc2_W, fc2_b, fc3_W, fc3_b):
    raise NotImplementedError("write your pallas kernel here")


# speedup vs baseline: 25.1898x; 25.1898x over previous
import functools

import jax
import jax.numpy as jnp
from jax import lax
from jax.experimental import pallas as pl
from jax.experimental.pallas import tpu as pltpu
from jax.experimental.pallas import tpu_sc as plsc

N = 51200
E = 819200
NC = 2
NS = 16
NW = NC * NS
ER = E // 128
RPW = ER // NW

_mesh = plsc.VectorSubcoreMesh(core_axis_name="c", subcore_axis_name="s")
_sc_params = pltpu.CompilerParams(use_tc_tiling_on_sc=False)

def _zero_fill(buf, nwords):
    zero16 = jnp.zeros((16,), jnp.float32)
    def body(i, _):
        buf[pl.ds(i * 16, 16)] = zero16
        return 0
    lax.fori_loop(0, nwords // 16, body, 0)


def _zero_fill2(buf, rows, width):
    zero16 = jnp.zeros((16,), jnp.float32)
    def body(i, _):
        for f2 in range(width // 16):
            buf[i, pl.ds(f2 * 16, 16)] = zero16
        return 0
    lax.fori_loop(0, rows, body, 0)


DK = 8


@functools.partial(
    pl.kernel,
    mesh=_mesh,
    compiler_params=_sc_params,
    out_type=jax.ShapeDtypeStruct((NC, N), jnp.float32),
    scratch_types=[
        pltpu.VMEM((N // NS,), jnp.float32),
        pltpu.VMEM((DK, 128), jnp.int32),
        pltpu.VMEM((DK, 128), jnp.float32),
        pltpu.VMEM_SHARED((N,), jnp.float32),
    ],
)
def _deg_sc(row_hbm, w_hbm, out_hbm, zbuf, row_v, w_v, deg_sh):
    c = lax.axis_index("c")
    s = lax.axis_index("s")
    wid = s * NC + c
    sl = N // NS
    _zero_fill(zbuf, sl)
    pltpu.sync_copy(zbuf, deg_sh.at[pl.ds(s * sl, sl)])
    plsc.subcore_barrier()

    def body(it, _):
        r0 = wid * RPW + it * DK
        pltpu.sync_copy(row_hbm.at[pl.ds(r0, DK)], row_v)
        pltpu.sync_copy(w_hbm.at[pl.ds(r0, DK)], w_v)
        for j in range(DK):
            pltpu.sync_copy(w_v.at[j], deg_sh.at[row_v.at[j]], add=True)
        return 0

    lax.fori_loop(0, RPW // DK, body, 0)
    plsc.subcore_barrier()
    pltpu.sync_copy(deg_sh.at[pl.ds(s * sl, sl)], zbuf)
    pltpu.sync_copy(zbuf, out_hbm.at[c, pl.ds(s * sl, sl)])


PK = 4


def _make_prop(F):
    zr = 100

    @functools.partial(
        pl.kernel,
        mesh=_mesh,
        compiler_params=_sc_params,
        out_type=jax.ShapeDtypeStruct((NC, N, F), jnp.float32),
        scratch_types=[
            pltpu.VMEM((zr, F), jnp.float32),
            pltpu.VMEM((PK, 128), jnp.int32),
            pltpu.VMEM((PK, 128), jnp.int32),
            pltpu.VMEM((PK, 128), jnp.float32),
            pltpu.VMEM((PK * 128, F), jnp.float32),
            pltpu.VMEM_SHARED((N, F), jnp.float32),
            pltpu.SemaphoreType.DMA,
        ],
    )
    def _prop(z_hbm, col_hbm, row_hbm, w_hbm, out_hbm,
              zbuf, col_v, row_v, w_v, rows_v, acc_sh, sem):
        c = lax.axis_index("c")
        s = lax.axis_index("s")
        wid = s * NC + c
        sl = N // NS

        _zero_fill2(zbuf, zr, F)
        for q in range(sl // zr):
            pltpu.sync_copy(zbuf, acc_sh.at[pl.ds(s * sl + q * zr, zr)])
        plsc.subcore_barrier()

        def body(it, _):
            r0 = wid * RPW + it * PK
            pltpu.sync_copy(col_hbm.at[pl.ds(r0, PK)], col_v)
            pltpu.sync_copy(row_hbm.at[pl.ds(r0, PK)], row_v)
            pltpu.sync_copy(w_hbm.at[pl.ds(r0, PK)], w_v)
            cps = [
                pltpu.async_copy(
                    z_hbm.at[col_v.at[j]],
                    rows_v.at[pl.ds(j * 128, 128)], sem)
                for j in range(PK)
            ]
            for cp in cps:
                cp.wait()
            for j in range(PK):
                def scale(j2, _, j=j):
                    w16 = w_v[j, pl.ds(j2 * 16, 16)]
                    for l in range(16):
                        r = j * 128 + j2 * 16 + l
                        w_s = w16[l]
                        for f2 in range(F // 16):
                            v = rows_v[r, pl.ds(f2 * 16, 16)]
                            rows_v[r, pl.ds(f2 * 16, 16)] = v * w_s
                    return 0
                lax.fori_loop(0, 8, scale, 0)
            for j in range(PK):
                pltpu.sync_copy(rows_v.at[pl.ds(j * 128, 128)],
                                acc_sh.at[row_v.at[j]], add=True)
            return 0

        lax.fori_loop(0, RPW // PK, body, 0)
        plsc.subcore_barrier()
        for q in range(sl // zr):
            pltpu.sync_copy(acc_sh.at[pl.ds(s * sl + q * zr, zr)], zbuf)
            pltpu.sync_copy(zbuf, out_hbm.at[c, pl.ds(s * sl + q * zr, zr)])

    return _prop


_prop32 = _make_prop(32)
_prop16 = _make_prop(16)


BN = 6400


def _dis_of(d0, d1):
    deg = d0 + d1
    return jnp.where(deg > 0, lax.rsqrt(jnp.where(deg > 0, deg, 1.0)), 0.0)


def _z1_body(x_ref, d0_ref, d1_ref, w_ref, o_ref):
    dis = _dis_of(d0_ref[...], d1_ref[...])
    y = jnp.dot(x_ref[...], w_ref[...], preferred_element_type=jnp.float32)
    o_ref[...] = y * dis


def _z1(x, d0, d1, W):
    Fin, F = W.shape
    grid = (N // BN,)
    return pl.pallas_call(
        _z1_body,
        grid=grid,
        in_specs=[
            pl.BlockSpec((BN, Fin), lambda i: (i, 0)),
            pl.BlockSpec((BN, 1), lambda i: (i, 0)),
            pl.BlockSpec((BN, 1), lambda i: (i, 0)),
            pl.BlockSpec((Fin, F), lambda i: (0, 0)),
        ],
        out_specs=pl.BlockSpec((BN, F), lambda i: (i, 0)),
        out_shape=jax.ShapeDtypeStruct((N, F), jnp.float32),
    )(x, d0, d1, W)


def _combine_body(h_ref, s0_ref, s1_ref, d0_ref, d1_ref, w0_ref, b_ref,
                  wn_ref, ho_ref, zo_ref):
    dis = _dis_of(d0_ref[...], d1_ref[...])
    S = (s0_ref[...] + s1_ref[...]) * dis
    hnew = jnp.maximum(
        jnp.dot(h_ref[...], w0_ref[...], preferred_element_type=jnp.float32)
        - S + b_ref[...], 0.0)
    ho_ref[...] = hnew
    zo_ref[...] = jnp.dot(hnew, wn_ref[...],
                          preferred_element_type=jnp.float32) * dis


def _combine(h, s0, s1, d0, d1, W0, b, Wn):
    Fin, F = W0.shape
    Fn = Wn.shape[1]
    grid = (N // BN,)
    return pl.pallas_call(
        _combine_body,
        grid=grid,
        in_specs=[
            pl.BlockSpec((BN, Fin), lambda i: (i, 0)),
            pl.BlockSpec((BN, F), lambda i: (i, 0)),
            pl.BlockSpec((BN, F), lambda i: (i, 0)),
            pl.BlockSpec((BN, 1), lambda i: (i, 0)),
            pl.BlockSpec((BN, 1), lambda i: (i, 0)),
            pl.BlockSpec((Fin, F), lambda i: (0, 0)),
            pl.BlockSpec((1, F), lambda i: (0, 0)),
            pl.BlockSpec((F, Fn), lambda i: (0, 0)),
        ],
        out_specs=[
            pl.BlockSpec((BN, F), lambda i: (i, 0)),
            pl.BlockSpec((BN, Fn), lambda i: (i, 0)),
        ],
        out_shape=[
            jax.ShapeDtypeStruct((N, F), jnp.float32),
            jax.ShapeDtypeStruct((N, Fn), jnp.float32),
        ],
    )(h, s0, s1, d0, d1, W0, b, Wn)


def _combine_last_body(h_ref, s0_ref, s1_ref, d0_ref, d1_ref, w0_ref, b_ref,
                       ho_ref):
    dis = _dis_of(d0_ref[...], d1_ref[...])
    S = (s0_ref[...] + s1_ref[...]) * dis
    ho_ref[...] = jnp.maximum(
        jnp.dot(h_ref[...], w0_ref[...], preferred_element_type=jnp.float32)
        - S + b_ref[...], 0.0)


def _combine_last(h, s0, s1, d0, d1, W0, b):
    Fin, F = W0.shape
    grid = (N // BN,)
    return pl.pallas_call(
        _combine_last_body,
        grid=grid,
        in_specs=[
            pl.BlockSpec((BN, Fin), lambda i: (i, 0)),
            pl.BlockSpec((BN, F), lambda i: (i, 0)),
            pl.BlockSpec((BN, F), lambda i: (i, 0)),
            pl.BlockSpec((BN, 1), lambda i: (i, 0)),
            pl.BlockSpec((BN, 1), lambda i: (i, 0)),
            pl.BlockSpec((Fin, F), lambda i: (0, 0)),
            pl.BlockSpec((1, F), lambda i: (0, 0)),
        ],
        out_specs=pl.BlockSpec((BN, F), lambda i: (i, 0)),
        out_shape=jax.ShapeDtypeStruct((N, F), jnp.float32),
    )(h, s0, s1, d0, d1, W0, b)


def _head_body(g_ref, w1_ref, b1_ref, w2_ref, b2_ref, w3_ref, b3_ref, o_ref):
    o = jnp.dot(g_ref[...], w1_ref[...],
                preferred_element_type=jnp.float32) + b1_ref[...]
    o = jnp.dot(o, w2_ref[...], preferred_element_type=jnp.float32) + b2_ref[...]
    o_ref[...] = jnp.dot(o, w3_ref[...],
                         preferred_element_type=jnp.float32) + b3_ref[...]


def _head(g, fc1_W, fc1_b, fc2_W, fc2_b, fc3_W, fc3_b):
    return pl.pallas_call(
        _head_body,
        out_shape=jax.ShapeDtypeStruct((g.shape[0], fc3_W.shape[1]),
                                       jnp.float32),
    )(g, fc1_W, fc1_b, fc2_W, fc2_b, fc3_W, fc3_b)


def kernel(x, edge_index, edge_weight, W1_0, W1_1, b1, W2_0, W2_1, b2,
           W3_0, W3_1, b3, fc1_W, fc1_b, fc2_W, fc2_b, fc3_W, fc3_b):
    row2d = edge_index[0].reshape(ER, 128)
    col2d = edge_index[1].reshape(ER, 128)
    w2d = edge_weight.reshape(ER, 128)

    degp = _deg_sc(row2d, w2d)
    d0 = degp[0].reshape(N, 1)
    d1 = degp[1].reshape(N, 1)

    z1 = _z1(x, d0, d1, W1_1)
    S1 = _prop32(z1, col2d, row2d, w2d)
    h1, z2 = _combine(x, S1[0], S1[1], d0, d1, W1_0, b1.reshape(1, -1), W2_1)
    S2 = _prop32(z2, col2d, row2d, w2d)
    h2, z3 = _combine(h1, S2[0], S2[1], d0, d1, W2_0, b2.reshape(1, -1), W3_1)
    S3 = _prop16(z3, col2d, row2d, w2d)
    h3 = _combine_last(h2, S3[0], S3[1], d0, d1, W3_0, b3.reshape(1, -1))

    g = h3.reshape(100, 512 * 16)
    return _head(g, fc1_W, fc1_b.reshape(1, -1), fc2_W, fc2_b.reshape(1, -1),
                 fc3_W, fc3_b.reshape(1, -1))


# speedup vs baseline: 33.7755x; 1.3408x over previous
import functools

import jax
import jax.numpy as jnp
from jax import lax
from jax.experimental import pallas as pl
from jax.experimental.pallas import tpu as pltpu
from jax.experimental.pallas import tpu_sc as plsc

N = 51200
E = 819200
NC = 2
NS = 16
NW = NC * NS
ER = E // 128
RPW = ER // NW

_mesh = plsc.VectorSubcoreMesh(core_axis_name="c", subcore_axis_name="s")
_sc_params = pltpu.CompilerParams(use_tc_tiling_on_sc=False)

def _zero_fill(buf, nwords):
    zero16 = jnp.zeros((16,), jnp.float32)
    def body(i, _):
        buf[pl.ds(i * 16, 16)] = zero16
        return 0
    lax.fori_loop(0, nwords // 16, body, 0)


def _zero_fill2(buf, rows, width):
    zero16 = jnp.zeros((16,), jnp.float32)
    def body(i, _):
        for f2 in range(width // 16):
            buf[i, pl.ds(f2 * 16, 16)] = zero16
        return 0
    lax.fori_loop(0, rows, body, 0)


DK = 8


@functools.partial(
    pl.kernel,
    mesh=_mesh,
    compiler_params=_sc_params,
    out_type=jax.ShapeDtypeStruct((NC, N), jnp.float32),
    scratch_types=[
        pltpu.VMEM((N // NS,), jnp.float32),
        pltpu.VMEM((DK, 128), jnp.int32),
        pltpu.VMEM((DK, 128), jnp.float32),
        pltpu.VMEM_SHARED((N,), jnp.float32),
    ],
)
def _deg_sc(row_hbm, w_hbm, out_hbm, zbuf, row_v, w_v, deg_sh):
    c = lax.axis_index("c")
    s = lax.axis_index("s")
    wid = s * NC + c
    sl = N // NS
    _zero_fill(zbuf, sl)
    pltpu.sync_copy(zbuf, deg_sh.at[pl.ds(s * sl, sl)])
    plsc.subcore_barrier()

    def body(it, _):
        r0 = wid * RPW + it * DK
        pltpu.sync_copy(row_hbm.at[pl.ds(r0, DK)], row_v)
        pltpu.sync_copy(w_hbm.at[pl.ds(r0, DK)], w_v)
        for j in range(DK):
            pltpu.sync_copy(w_v.at[j], deg_sh.at[row_v.at[j]], add=True)
        return 0

    lax.fori_loop(0, RPW // DK, body, 0)
    plsc.subcore_barrier()
    pltpu.sync_copy(deg_sh.at[pl.ds(s * sl, sl)], zbuf)
    pltpu.sync_copy(zbuf, out_hbm.at[c, pl.ds(s * sl, sl)])


PK = 2
NSLAB = 4


def _make_prop(F):
    zr = 100
    CH = RPW // PK

    @functools.partial(
        pl.kernel,
        mesh=_mesh,
        compiler_params=_sc_params,
        out_type=jax.ShapeDtypeStruct((NC, N, F), jnp.float32),
        scratch_types=[
            pltpu.VMEM((zr, F), jnp.float32),
            pltpu.VMEM((NSLAB, PK, 128), jnp.int32),
            pltpu.VMEM((NSLAB, PK, 128), jnp.int32),
            pltpu.VMEM((NSLAB, PK, 128), jnp.float32),
            pltpu.VMEM((2, PK * 128, F), jnp.float32),
            pltpu.VMEM_SHARED((N, F), jnp.float32),
            pltpu.SemaphoreType.DMA((NSLAB,)),
            pltpu.SemaphoreType.DMA((2,)),
            pltpu.SemaphoreType.DMA((2,)),
        ],
    )
    def _prop(z_hbm, col_hbm, row_hbm, w_hbm, out_hbm,
              zbuf, col_v, row_v, w_v, rows_v, acc_sh, semsl, semg, semsc):
        c = lax.axis_index("c")
        s = lax.axis_index("s")
        wid = s * NC + c
        sl = N // NS
        base = wid * RPW

        _zero_fill2(zbuf, zr, F)
        for q in range(sl // zr):
            pltpu.sync_copy(zbuf, acc_sh.at[pl.ds(s * sl + q * zr, zr)])
        plsc.subcore_barrier()

        def slab_start(ci, sb):
            r0 = base + ci * PK
            pltpu.async_copy(col_hbm.at[pl.ds(r0, PK)], col_v.at[sb],
                             semsl.at[sb])
            pltpu.async_copy(row_hbm.at[pl.ds(r0, PK)], row_v.at[sb],
                             semsl.at[sb])
            pltpu.async_copy(w_hbm.at[pl.ds(r0, PK)], w_v.at[sb],
                             semsl.at[sb])

        def slab_wait(ci, sb):
            r0 = base + ci * PK
            pltpu.make_async_copy(col_hbm.at[pl.ds(r0, PK)], col_v.at[sb],
                                  semsl.at[sb]).wait()
            pltpu.make_async_copy(row_hbm.at[pl.ds(r0, PK)], row_v.at[sb],
                                  semsl.at[sb]).wait()
            pltpu.make_async_copy(w_hbm.at[pl.ds(r0, PK)], w_v.at[sb],
                                  semsl.at[sb]).wait()

        def gather_start(sb, rb):
            for j in range(PK):
                pltpu.async_copy(z_hbm.at[col_v.at[sb, j]],
                                 rows_v.at[rb, pl.ds(j * 128, 128)],
                                 semg.at[rb])

        def gather_wait(sb, rb):
            for j in range(PK):
                pltpu.make_async_copy(z_hbm.at[col_v.at[sb, j]],
                                      rows_v.at[rb, pl.ds(j * 128, 128)],
                                      semg.at[rb]).wait()

        def scale(rb, sb):
            for j in range(PK):
                def sbody(j2, _, j=j):
                    w16 = w_v[sb, j, pl.ds(j2 * 16, 16)]
                    for l in range(16):
                        r = j * 128 + j2 * 16 + l
                        w_s = w16[l]
                        for f2 in range(F // 16):
                            v = rows_v[rb, r, pl.ds(f2 * 16, 16)]
                            rows_v[rb, r, pl.ds(f2 * 16, 16)] = v * w_s
                    return 0
                lax.fori_loop(0, 8, sbody, 0)

        def scat_start(rb, sb):
            for j in range(PK):
                pltpu.async_copy(rows_v.at[rb, pl.ds(j * 128, 128)],
                                 acc_sh.at[row_v.at[sb, j]],
                                 semsc.at[rb], add=True)

        def scat_wait(rb, sb):
            for j in range(PK):
                pltpu.make_async_copy(rows_v.at[rb, pl.ds(j * 128, 128)],
                                      acc_sh.at[row_v.at[sb, j]],
                                      semsc.at[rb]).wait()

        def chunk(ci, cs, first=False, gnext=True, snext=True):
            rb, sb = cs & 1, cs % NSLAB
            gather_wait(sb, rb)
            scale(rb, sb)
            scat_start(rb, sb)
            if not first:
                scat_wait((cs - 1) & 1, (cs - 1) % NSLAB)
            if gnext:
                slab_wait(ci + 1, (cs + 1) % NSLAB)
                gather_start((cs + 1) % NSLAB, (cs + 1) & 1)
            if snext:
                slab_start(ci + 3, (cs + 3) % NSLAB)

        slab_start(0, 0)
        slab_start(1, 1)
        slab_start(2, 2)
        slab_wait(0, 0)
        gather_start(0, 0)
        chunk(0, 0, first=True)
        chunk(1, 1)
        chunk(2, 2)
        chunk(3, 3)

        def quad(t, _):
            c0 = 4 + t * 4
            for o in range(4):
                chunk(c0 + o, o)
            return 0
        lax.fori_loop(0, (CH - 8) // 4, quad, 0)

        chunk(CH - 4, 0)
        chunk(CH - 3, 1, snext=False)
        chunk(CH - 2, 2, snext=False)
        chunk(CH - 1, 3, gnext=False, snext=False)
        scat_wait(1, 3)

        plsc.subcore_barrier()
        for q in range(sl // zr):
            pltpu.sync_copy(acc_sh.at[pl.ds(s * sl + q * zr, zr)], zbuf)
            pltpu.sync_copy(zbuf, out_hbm.at[c, pl.ds(s * sl + q * zr, zr)])

    return _prop


_prop32 = _make_prop(32)
_prop16 = _make_prop(16)


BN = 6400


def _dis_of(d0, d1):
    deg = d0 + d1
    return jnp.where(deg > 0, lax.rsqrt(jnp.where(deg > 0, deg, 1.0)), 0.0)


def _z1_body(x_ref, dp_ref, w_ref, o_ref, dis_ref):
    dis = _dis_of(dp_ref[0], dp_ref[1])
    y = jnp.dot(x_ref[...], w_ref[...], preferred_element_type=jnp.float32)
    o_ref[...] = y * dis
    dis_ref[...] = dis


def _z1(x, degp, W):
    Fin, F = W.shape
    grid = (N // BN,)
    return pl.pallas_call(
        _z1_body,
        grid=grid,
        in_specs=[
            pl.BlockSpec((BN, Fin), lambda i: (i, 0)),
            pl.BlockSpec((2, BN, 1), lambda i: (0, i, 0)),
            pl.BlockSpec((Fin, F), lambda i: (0, 0)),
        ],
        out_specs=[
            pl.BlockSpec((BN, F), lambda i: (i, 0)),
            pl.BlockSpec((BN, 1), lambda i: (i, 0)),
        ],
        out_shape=[
            jax.ShapeDtypeStruct((N, F), jnp.float32),
            jax.ShapeDtypeStruct((N, 1), jnp.float32),
        ],
    )(x, degp, W)


def _combine_body(h_ref, S_ref, dis_ref, w0_ref, b_ref, wn_ref,
                  ho_ref, zo_ref):
    dis = dis_ref[...]
    S = (S_ref[0] + S_ref[1]) * dis
    hnew = jnp.maximum(
        jnp.dot(h_ref[...], w0_ref[...], preferred_element_type=jnp.float32)
        - S + b_ref[...], 0.0)
    ho_ref[...] = hnew
    zo_ref[...] = jnp.dot(hnew, wn_ref[...],
                          preferred_element_type=jnp.float32) * dis


def _combine(h, S, dis, W0, b, Wn):
    Fin, F = W0.shape
    Fn = Wn.shape[1]
    grid = (N // BN,)
    return pl.pallas_call(
        _combine_body,
        grid=grid,
        in_specs=[
            pl.BlockSpec((BN, Fin), lambda i: (i, 0)),
            pl.BlockSpec((2, BN, F), lambda i: (0, i, 0)),
            pl.BlockSpec((BN, 1), lambda i: (i, 0)),
            pl.BlockSpec((Fin, F), lambda i: (0, 0)),
            pl.BlockSpec((1, F), lambda i: (0, 0)),
            pl.BlockSpec((F, Fn), lambda i: (0, 0)),
        ],
        out_specs=[
            pl.BlockSpec((BN, F), lambda i: (i, 0)),
            pl.BlockSpec((BN, Fn), lambda i: (i, 0)),
        ],
        out_shape=[
            jax.ShapeDtypeStruct((N, F), jnp.float32),
            jax.ShapeDtypeStruct((N, Fn), jnp.float32),
        ],
    )(h, S, dis, W0, b, Wn)


def _combine_last_body(h_ref, S_ref, dis_ref, w0_ref, b_ref, ho_ref):
    S = (S_ref[0] + S_ref[1]) * dis_ref[...]
    ho_ref[...] = jnp.maximum(
        jnp.dot(h_ref[...], w0_ref[...], preferred_element_type=jnp.float32)
        - S + b_ref[...], 0.0)


def _combine_last(h, S, dis, W0, b):
    Fin, F = W0.shape
    grid = (N // BN,)
    return pl.pallas_call(
        _combine_last_body,
        grid=grid,
        in_specs=[
            pl.BlockSpec((BN, Fin), lambda i: (i, 0)),
            pl.BlockSpec((2, BN, F), lambda i: (0, i, 0)),
            pl.BlockSpec((BN, 1), lambda i: (i, 0)),
            pl.BlockSpec((Fin, F), lambda i: (0, 0)),
            pl.BlockSpec((1, F), lambda i: (0, 0)),
        ],
        out_specs=pl.BlockSpec((BN, F), lambda i: (i, 0)),
        out_shape=jax.ShapeDtypeStruct((N, F), jnp.float32),
    )(h, S, dis, W0, b)


def _head_body(g_ref, w1_ref, b1_ref, w2_ref, b2_ref, w3_ref, b3_ref, o_ref):
    o = jnp.dot(g_ref[...], w1_ref[...],
                preferred_element_type=jnp.float32) + b1_ref[...]
    o = jnp.dot(o, w2_ref[...], preferred_element_type=jnp.float32) + b2_ref[...]
    o_ref[...] = jnp.dot(o, w3_ref[...],
                         preferred_element_type=jnp.float32) + b3_ref[...]


def _head(g, fc1_W, fc1_b, fc2_W, fc2_b, fc3_W, fc3_b):
    return pl.pallas_call(
        _head_body,
        out_shape=jax.ShapeDtypeStruct((g.shape[0], fc3_W.shape[1]),
                                       jnp.float32),
    )(g, fc1_W, fc1_b, fc2_W, fc2_b, fc3_W, fc3_b)


def kernel(x, edge_index, edge_weight, W1_0, W1_1, b1, W2_0, W2_1, b2,
           W3_0, W3_1, b3, fc1_W, fc1_b, fc2_W, fc2_b, fc3_W, fc3_b):
    row2d = edge_index[0].reshape(ER, 128)
    col2d = edge_index[1].reshape(ER, 128)
    w2d = edge_weight.reshape(ER, 128)

    degp = _deg_sc(row2d, w2d).reshape(NC, N, 1)

    z1, dis = _z1(x, degp, W1_1)
    S1 = _prop32(z1, col2d, row2d, w2d)
    h1, z2 = _combine(x, S1, dis, W1_0, b1.reshape(1, -1), W2_1)
    S2 = _prop32(z2, col2d, row2d, w2d)
    h2, z3 = _combine(h1, S2, dis, W2_0, b2.reshape(1, -1), W3_1)
    S3 = _prop16(z3, col2d, row2d, w2d)
    h3 = _combine_last(h2, S3, dis, W3_0, b3.reshape(1, -1))

    g = h3.reshape(100, 512 * 16)
    return _head(g, fc1_W, fc1_b.reshape(1, -1), fc2_W, fc2_b.reshape(1, -1),
                 fc3_W, fc3_b.reshape(1, -1))
